# baseline (speedup 1.0000x reference)
import functools

import jax
import jax.numpy as jnp
from jax import lax
from jax.experimental import pallas as pl
from jax.experimental.pallas import tpu as pltpu
from jax.experimental.pallas import tpu_sc as plsc

DIM = 64
SEQ = 128
EPS = 1e-5
NC = 2
NS = 16
NW = NC * NS
CHUNK = 128
NBUF = 4
OBUF = 2


def _rsqrt(x):
    i = plsc.bitcast(x, jnp.int32)
    i = jnp.int32(0x5F3759DF) - lax.shift_right_logical(i, 1)
    y = plsc.bitcast(i, jnp.float32)
    hx = 0.5 * x
    for _ in range(2):
        y = y * (1.5 - hx * y * y)
    return y


def _lane_sum(v):
    idx = lax.iota(jnp.int32, 16)
    dnums = lax.GatherDimensionNumbers(
        offset_dims=(), collapsed_slice_dims=(0,), start_index_map=(0,))
    for k in (8, 4, 2, 1):
        perm = lax.bitwise_xor(idx, jnp.int32(k))
        v = v + lax.gather(v, perm[:, None], dnums, slice_sizes=(1,),
                           mode=lax.GatherScatterMode.PROMISE_IN_BOUNDS)
    return v


def _sc_body(total_rows, ids_hbm, pairs_hbm, pos_hbm, gamma_hbm, beta_hbm,
             out_hbm, idx_v, idx2_v, rows_v, stage_v, pos_v, gamma_v,
             beta_v, in_sems, out_sems):
    wid = lax.axis_index("s") * NC + lax.axis_index("c")
    rows_per_w = total_rows // NW
    base = pl.multiple_of(wid * rows_per_w, CHUNK)
    seq_base = base // SEQ

    pltpu.sync_copy(ids_hbm.at[pl.ds(base, rows_per_w)],
                    idx_v.at[pl.ds(0, rows_per_w)])
    pltpu.sync_copy(pos_hbm, pos_v)
    pltpu.sync_copy(gamma_hbm, gamma_v)
    pltpu.sync_copy(beta_hbm, beta_v)

    @plsc.parallel_loop(0, rows_per_w // 16, unroll=8)
    def _half(i):
        idx2_v[pl.ds(i * 16, 16)] = lax.shift_right_logical(
            idx_v[pl.ds(i * 16, 16)], 1)

    g = [gamma_v[pl.ds(16 * j, 16)] for j in range(4)]
    b = [beta_v[pl.ds(16 * j, 16)] for j in range(4)]
    row_ids = [lax.iota(jnp.int32, 16) + 16 * j for j in range(4)]
    inv_d = jnp.float32(1.0 / DIM)

    nchunks = rows_per_w // CHUNK

    def start_gather(c, buf):
        off = pl.multiple_of(c * CHUNK, CHUNK)
        pltpu.async_copy(pairs_hbm.at[idx2_v.at[pl.ds(off, CHUNK)]],
                         rows_v.at[buf], in_sems.at[buf])

    def wait_dma(dst, sem):
        pltpu.make_async_copy(pairs_hbm.at[pl.ds(0, dst.shape[0])], dst,
                              sem).wait()

    start_gather(0, 0)
    start_gather(1, 1)

    def group_body(grp, _):
        for bi in range(NBUF):
            c = grp * NBUF + bi
            oi = bi % OBUF
            buf = rows_v.at[bi]
            stage = stage_v.at[oi]
            off = pl.multiple_of(c * CHUNK, CHUNK)

            wait_dma(buf, in_sems.at[bi])

            @pl.when(c >= OBUF)
            def _():
                wait_dma(stage, out_sems.at[oi])

            @plsc.parallel_loop(0, CHUNK, unroll=4)
            def row_body(r):
                par = idx_v[pl.ds(off + r, 16)][0] & 1
                d0 = par * DIM
                x = [buf[r, pl.ds(d0 + 16 * j, 16)]
                     + pos_v[r, pl.ds(16 * j, 16)] for j in range(4)]
                s = _lane_sum(x[0] + x[1] + x[2] + x[3])
                q = _lane_sum(x[0] * x[0] + x[1] * x[1]
                              + x[2] * x[2] + x[3] * x[3])
                mean = s * inv_d
                var = q * inv_d - mean * mean
                rstd = _rsqrt(var + EPS)
                col = jnp.full((16,), r, jnp.int32)
                for j in range(4):
                    y = (x[j] - mean) * rstd * g[j] + b[j]
                    plsc.store_scatter(stage, [row_ids[j], col], y)

            pltpu.async_copy(stage,
                             out_hbm.at[pl.ds((seq_base + c) * DIM, DIM)],
                             out_sems.at[oi])

            nxt = c + 2

            @pl.when(nxt < nchunks)
            def _():
                start_gather(nxt, (bi + 2) % NBUF)
        return 0

    lax.fori_loop(0, nchunks // NBUF, group_body, 0)

    for oi in range(OBUF):
        wait_dma(stage_v.at[oi], out_sems.at[oi])


def kernel(input_ids, token_table, pos_table, gamma, beta):
    batch, seq = input_ids.shape
    total_rows = batch * seq
    vocab = token_table.shape[0]
    ids_flat = input_ids.reshape(total_rows).astype(jnp.int32)
    pairs = token_table.reshape(vocab // 2, 2 * DIM)
    rows_per_w = total_rows // NW

    mesh = plsc.VectorSubcoreMesh(core_axis_name="c", subcore_axis_name="s")
    out_t = pl.kernel(
        functools.partial(_sc_body, total_rows),
        out_type=jax.ShapeDtypeStruct((batch * DIM, SEQ), jnp.float32),
        mesh=mesh,
        compiler_params=pltpu.CompilerParams(
            needs_layout_passes=False, use_tc_tiling_on_sc=False),
        scratch_types=[
            pltpu.VMEM((rows_per_w + 16,), jnp.int32),
            pltpu.VMEM((rows_per_w,), jnp.int32),
            pltpu.VMEM((NBUF, CHUNK, 2 * DIM), jnp.float32),
            pltpu.VMEM((OBUF, DIM, SEQ), jnp.float32),
            pltpu.VMEM((SEQ, DIM), jnp.float32),
            pltpu.VMEM((DIM,), jnp.float32),
            pltpu.VMEM((DIM,), jnp.float32),
            pltpu.SemaphoreType.DMA((NBUF,)),
            pltpu.SemaphoreType.DMA((OBUF,)),
        ],
    )(ids_flat, pairs, pos_table, gamma, beta)
    return out_t.reshape(batch, DIM, SEQ).transpose(0, 2, 1)


# speedup vs baseline: 1.0534x; 1.0534x over previous
import functools

import jax
import jax.numpy as jnp
from jax import lax
from jax.experimental import pallas as pl
from jax.experimental.pallas import tpu as pltpu
from jax.experimental.pallas import tpu_sc as plsc

DIM = 64
SEQ = 128
EPS = 1e-5
NC = 2
NS = 16
NW = NC * NS
CHUNK = 128
NBUF = 4
OBUF = 2


def _rsqrt(x):
    i = plsc.bitcast(x, jnp.int32)
    i = jnp.int32(0x5F3759DF) - lax.shift_right_logical(i, 1)
    y = plsc.bitcast(i, jnp.float32)
    hx = 0.5 * x
    for _ in range(2):
        y = y * (1.5 - hx * y * y)
    return y


def _lane_sum(v):
    idx = lax.iota(jnp.int32, 16)
    dnums = lax.GatherDimensionNumbers(
        offset_dims=(), collapsed_slice_dims=(0,), start_index_map=(0,))
    for k in (8, 4, 2, 1):
        perm = lax.bitwise_xor(idx, jnp.int32(k))
        v = v + lax.gather(v, perm[:, None], dnums, slice_sizes=(1,),
                           mode=lax.GatherScatterMode.PROMISE_IN_BOUNDS)
    return v


def _sc_body(total_rows, ids_hbm, table_hbm, pos_hbm, gamma_hbm, beta_hbm,
             out_hbm, idx_v, rows_v, stage_v, pos_v, gamma_v, beta_v,
             in_sems, out_sems):
    wid = lax.axis_index("s") * NC + lax.axis_index("c")
    rows_per_w = total_rows // NW
    base = pl.multiple_of(wid * rows_per_w, CHUNK)
    seq_base = base // SEQ

    pltpu.sync_copy(ids_hbm.at[pl.ds(base, rows_per_w)], idx_v)
    pltpu.sync_copy(pos_hbm, pos_v)
    pltpu.sync_copy(gamma_hbm, gamma_v)
    pltpu.sync_copy(beta_hbm, beta_v)

    g = [gamma_v[pl.ds(16 * j, 16)] for j in range(4)]
    b = [beta_v[pl.ds(16 * j, 16)] for j in range(4)]
    row_ids = [lax.iota(jnp.int32, 16) + 16 * j for j in range(4)]
    inv_d = jnp.float32(1.0 / DIM)

    nchunks = rows_per_w // CHUNK

    def start_gather(c, buf):
        off = pl.multiple_of(c * CHUNK, CHUNK)
        pltpu.async_copy(table_hbm.at[idx_v.at[pl.ds(off, CHUNK)]],
                         rows_v.at[buf], in_sems.at[buf])

    def wait_dma(dst, sem):
        pltpu.make_async_copy(table_hbm.at[pl.ds(0, dst.shape[0])], dst,
                              sem).wait()

    start_gather(0, 0)
    start_gather(1, 1)

    def group_body(grp, _):
        for bi in range(NBUF):
            c = grp * NBUF + bi
            oi = bi % OBUF
            buf = rows_v.at[bi]
            stage = stage_v.at[oi]

            wait_dma(buf, in_sems.at[bi])

            @pl.when(c >= OBUF)
            def _():
                wait_dma(stage, out_sems.at[oi])

            @plsc.parallel_loop(0, CHUNK, unroll=4)
            def row_body(r):
                x = [buf[r, pl.ds(16 * j, 16)] + pos_v[r, pl.ds(16 * j, 16)]
                     for j in range(4)]
                s = _lane_sum(x[0] + x[1] + x[2] + x[3])
                q = _lane_sum(x[0] * x[0] + x[1] * x[1]
                              + x[2] * x[2] + x[3] * x[3])
                mean = s * inv_d
                var = q * inv_d - mean * mean
                rstd = _rsqrt(var + EPS)
                col = jnp.full((16,), r, jnp.int32)
                for j in range(4):
                    y = (x[j] - mean) * rstd * g[j] + b[j]
                    plsc.store_scatter(stage, [row_ids[j], col], y)

            pltpu.async_copy(stage,
                             out_hbm.at[pl.ds((seq_base + c) * DIM, DIM)],
                             out_sems.at[oi])

            nxt = c + 2

            @pl.when(nxt < nchunks)
            def _():
                start_gather(nxt, (bi + 2) % NBUF)
        return 0

    lax.fori_loop(0, nchunks // NBUF, group_body, 0)

    for oi in range(OBUF):
        wait_dma(stage_v.at[oi], out_sems.at[oi])


def kernel(input_ids, token_table, pos_table, gamma, beta):
    batch, seq = input_ids.shape
    total_rows = batch * seq
    ids_flat = input_ids.reshape(total_rows).astype(jnp.int32)
    rows_per_w = total_rows // NW

    mesh = plsc.VectorSubcoreMesh(core_axis_name="c", subcore_axis_name="s")
    out_t = pl.kernel(
        functools.partial(_sc_body, total_rows),
        out_type=jax.ShapeDtypeStruct((batch * DIM, SEQ), jnp.float32),
        mesh=mesh,
        compiler_params=pltpu.CompilerParams(
            needs_layout_passes=False, use_tc_tiling_on_sc=False),
        scratch_types=[
            pltpu.VMEM((rows_per_w,), jnp.int32),
            pltpu.VMEM((NBUF, CHUNK, DIM), jnp.float32),
            pltpu.VMEM((OBUF, DIM, SEQ), jnp.float32),
            pltpu.VMEM((SEQ, DIM), jnp.float32),
            pltpu.VMEM((DIM,), jnp.float32),
            pltpu.VMEM((DIM,), jnp.float32),
            pltpu.SemaphoreType.DMA((NBUF,)),
            pltpu.SemaphoreType.DMA((OBUF,)),
        ],
    )(ids_flat, token_table, pos_table, gamma, beta)
    return out_t.reshape(batch, DIM, SEQ).transpose(0, 2, 1)


# speedup vs baseline: 1.4614x; 1.3873x over previous
import functools

import jax
import jax.numpy as jnp
from jax import lax
from jax.experimental import pallas as pl
from jax.experimental.pallas import tpu as pltpu
from jax.experimental.pallas import tpu_sc as plsc

DIM = 64
SEQ = 128
EPS = 1e-5
NC = 2
NS = 16
NW = NC * NS
CHUNK = 128
NBUF = 4
OBUF = 2
SPAD = SEQ + 1


def _rsqrt(x):
    i = plsc.bitcast(x, jnp.int32)
    i = jnp.int32(0x5F3759DF) - lax.shift_right_logical(i, 1)
    y = plsc.bitcast(i, jnp.float32)
    hx = 0.5 * x
    for _ in range(2):
        y = y * (1.5 - hx * y * y)
    return y


def _lane_sum(v):
    idx = lax.iota(jnp.int32, 16)
    dnums = lax.GatherDimensionNumbers(
        offset_dims=(), collapsed_slice_dims=(0,), start_index_map=(0,))
    for k in (8, 4, 2, 1):
        perm = lax.bitwise_xor(idx, jnp.int32(k))
        v = v + lax.gather(v, perm[:, None], dnums, slice_sizes=(1,),
                           mode=lax.GatherScatterMode.PROMISE_IN_BOUNDS)
    return v


def _sc_body(total_rows, ids_hbm, table_hbm, pos_hbm, gamma_hbm, beta_hbm,
             out_hbm, idx_v, rows_v, stage_v, pos_v, gamma_v, beta_v,
             in_sems, out_sems):
    wid = lax.axis_index("s") * NC + lax.axis_index("c")
    rows_per_w = total_rows // NW
    base = pl.multiple_of(wid * rows_per_w, CHUNK)
    seq_base = base // SEQ

    pltpu.sync_copy(ids_hbm.at[pl.ds(base, rows_per_w)], idx_v)
    pltpu.sync_copy(pos_hbm, pos_v)
    pltpu.sync_copy(gamma_hbm, gamma_v)
    pltpu.sync_copy(beta_hbm, beta_v)

    g = [gamma_v[pl.ds(16 * j, 16)] for j in range(4)]
    b = [beta_v[pl.ds(16 * j, 16)] for j in range(4)]
    row_ids = [lax.iota(jnp.int32, 16) + 16 * j for j in range(4)]
    inv_d = jnp.float32(1.0 / DIM)

    nchunks = rows_per_w // CHUNK

    def start_gather(c, buf):
        off = pl.multiple_of(c * CHUNK, CHUNK)
        pltpu.async_copy(table_hbm.at[idx_v.at[pl.ds(off, CHUNK)]],
                         rows_v.at[buf], in_sems.at[buf])

    def wait_dma(dst, sem):
        pltpu.make_async_copy(table_hbm.at[pl.ds(0, dst.shape[0])], dst,
                              sem).wait()

    def wait_out(oi):
        pltpu.make_async_copy(
            out_hbm.at[pl.ds(0, DIM)],
            stage_v.at[oi].at[:, pl.ds(0, SEQ)], out_sems.at[oi]).wait()

    start_gather(0, 0)
    start_gather(1, 1)

    def group_body(grp, _):
        for bi in range(NBUF):
            c = grp * NBUF + bi
            oi = bi % OBUF
            buf = rows_v.at[bi]
            stage = stage_v.at[oi]

            wait_dma(buf, in_sems.at[bi])

            @pl.when(c >= OBUF)
            def _():
                wait_out(oi)

            @plsc.parallel_loop(0, CHUNK, unroll=4)
            def row_body(r):
                x = [buf[r, pl.ds(16 * j, 16)] + pos_v[r, pl.ds(16 * j, 16)]
                     for j in range(4)]
                s = _lane_sum(x[0] + x[1] + x[2] + x[3])
                q = _lane_sum(x[0] * x[0] + x[1] * x[1]
                              + x[2] * x[2] + x[3] * x[3])
                mean = s * inv_d
                var = q * inv_d - mean * mean
                rstd = _rsqrt(var + EPS)
                col = jnp.full((16,), r, jnp.int32)
                for j in range(4):
                    y = (x[j] - mean) * rstd * g[j] + b[j]
                    plsc.store_scatter(stage, [row_ids[j], col], y)

            pltpu.async_copy(stage.at[:, pl.ds(0, SEQ)],
                             out_hbm.at[pl.ds((seq_base + c) * DIM, DIM)],
                             out_sems.at[oi])

            nxt = c + 2

            @pl.when(nxt < nchunks)
            def _():
                start_gather(nxt, (bi + 2) % NBUF)
        return 0

    lax.fori_loop(0, nchunks // NBUF, group_body, 0)

    for oi in range(OBUF):
        wait_out(oi)


def kernel(input_ids, token_table, pos_table, gamma, beta):
    batch, seq = input_ids.shape
    total_rows = batch * seq
    ids_flat = input_ids.reshape(total_rows).astype(jnp.int32)
    rows_per_w = total_rows // NW

    mesh = plsc.VectorSubcoreMesh(core_axis_name="c", subcore_axis_name="s")
    out_t = pl.kernel(
        functools.partial(_sc_body, total_rows),
        out_type=jax.ShapeDtypeStruct((batch * DIM, SEQ), jnp.float32),
        mesh=mesh,
        compiler_params=pltpu.CompilerParams(
            needs_layout_passes=False, use_tc_tiling_on_sc=False),
        scratch_types=[
            pltpu.VMEM((rows_per_w,), jnp.int32),
            pltpu.VMEM((NBUF, CHUNK, DIM), jnp.float32),
            pltpu.VMEM((OBUF, DIM, SPAD), jnp.float32),
            pltpu.VMEM((SEQ, DIM), jnp.float32),
            pltpu.VMEM((DIM,), jnp.float32),
            pltpu.VMEM((DIM,), jnp.float32),
            pltpu.SemaphoreType.DMA((NBUF,)),
            pltpu.SemaphoreType.DMA((OBUF,)),
        ],
    )(ids_flat, token_table, pos_table, gamma, beta)
    return out_t.reshape(batch, DIM, SEQ).transpose(0, 2, 1)
